# baseline (speedup 1.0000x reference)
import functools

import jax
import jax.numpy as jnp
from jax import lax
from jax.experimental import pallas as pl
from jax.experimental.pallas import tpu as pltpu
from jax.experimental.pallas import tpu_sc as plsc

NC = 2
NS = 16
NW = NC * NS
L = 16

NEG_INF = float("-inf")


def _pq_body(x_ref, w1_ref, b1_ref, p_ref, q_ref):
    x = x_ref[...]
    wa = w1_ref[0:128, :]
    wb = w1_ref[128:256, :]
    q = jnp.dot(x, wb, preferred_element_type=jnp.float32)
    p = jnp.dot(x, wa, preferred_element_type=jnp.float32) - q + b1_ref[...]
    p_ref[...] = p
    q_ref[...] = q


def _tc_pq(x, W1, b1):
    n, d = x.shape
    blk = 1000
    grid = n // blk
    return pl.pallas_call(
        _pq_body,
        grid=(grid,),
        in_specs=[
            pl.BlockSpec((blk, d), lambda i: (i, 0)),
            pl.BlockSpec((2 * d, d), lambda i: (0, 0)),
            pl.BlockSpec((1, d), lambda i: (0, 0)),
        ],
        out_specs=[
            pl.BlockSpec((blk, d), lambda i: (i, 0)),
            pl.BlockSpec((blk, d), lambda i: (i, 0)),
        ],
        out_shape=[
            jax.ShapeDtypeStruct((n, d), jnp.float32),
            jax.ShapeDtypeStruct((n, d), jnp.float32),
        ],
    )(x, W1, b1.reshape(1, d))


def _sc_gather_add(dst, src, P, Q):
    e = dst.shape[0]
    d = P.shape[1]
    ept = e // NW
    chunk = 400
    nch = ept // chunk

    mesh = plsc.VectorSubcoreMesh(core_axis_name="c", subcore_axis_name="s")

    @functools.partial(
        pl.kernel,
        out_type=jax.ShapeDtypeStruct((e, d), jnp.float32),
        mesh=mesh,
        scratch_types=[
            pltpu.VMEM((chunk,), jnp.int32),
            pltpu.VMEM((chunk,), jnp.int32),
            pltpu.VMEM((chunk, d), jnp.float32),
            pltpu.VMEM((chunk, d), jnp.float32),
            pltpu.SemaphoreType.DMA,
        ],
    )
    def k(dst_hbm, src_hbm, p_hbm, q_hbm, g_hbm, didx, sidx, bufa, bufb, sem):
        wid = lax.axis_index("s") * NC + lax.axis_index("c")
        tile_base = wid * ept

        @pl.loop(0, nch)
        def _chunk(i):
            base = tile_base + i * chunk
            pltpu.sync_copy(dst_hbm.at[pl.ds(base, chunk)], didx)
            pltpu.sync_copy(src_hbm.at[pl.ds(base, chunk)], sidx)
            pltpu.async_copy(p_hbm.at[didx], bufa, sem).wait()
            pltpu.async_copy(q_hbm.at[sidx], bufb, sem).wait()

            @pl.loop(0, chunk)
            def _row(r):
                for j in range(d // L):
                    sl = pl.ds(j * L, L)
                    bufa[r, sl] = bufa[r, sl] + bufb[r, sl]

            pltpu.sync_copy(bufa, g_hbm.at[pl.ds(base, chunk)])

    return k(dst, src, P, Q)


def _msg_body(g_ref, w2_ref, b2_ref, o_ref):
    g = g_ref[...]
    h = jnp.maximum(g, 0.01 * g)
    o_ref[...] = jnp.dot(h, w2_ref[...], preferred_element_type=jnp.float32) + b2_ref[...]


def _tc_msg(G, W2, b2):
    e, d = G.shape
    blk = 2000
    grid = e // blk
    return pl.pallas_call(
        _msg_body,
        grid=(grid,),
        in_specs=[
            pl.BlockSpec((blk, d), lambda i: (i, 0)),
            pl.BlockSpec((d, d), lambda i: (0, 0)),
            pl.BlockSpec((1, d), lambda i: (0, 0)),
        ],
        out_specs=pl.BlockSpec((blk, d), lambda i: (i, 0)),
        out_shape=jax.ShapeDtypeStruct((e, d), jnp.float32),
    )(G, W2, b2.reshape(1, d))


def _sc_scatter_max(dst, MSG, n):
    e, d = MSG.shape
    rt = 320
    rt_last = n - (NW - 1) * rt
    cd = 2000
    nch = e // cd
    gb = 256
    nvec = cd // L

    mesh = plsc.VectorSubcoreMesh(core_axis_name="c", subcore_axis_name="s")

    @functools.partial(
        pl.kernel,
        out_type=jax.ShapeDtypeStruct((n, d), jnp.float32),
        mesh=mesh,
        scratch_types=[
            pltpu.VMEM((rt, d), jnp.float32),
            pltpu.VMEM((cd,), jnp.int32),
            pltpu.VMEM((cd + L,), jnp.int32),
            pltpu.VMEM((gb,), jnp.int32),
            pltpu.VMEM((gb, d), jnp.float32),
            pltpu.SemaphoreType.DMA,
        ],
    )
    def k(dst_hbm, msg_hbm, agg_hbm, acc, dbuf, mval, gidx, rows, sem):
        wid = lax.axis_index("s") * NC + lax.axis_index("c")
        node_base = wid * rt

        ninf = jnp.full((L,), NEG_INF, dtype=jnp.float32)
        iota = lax.iota(jnp.int32, L)

        @pl.loop(0, rt)
        def _init(r):
            for j in range(d // L):
                acc[r, pl.ds(j * L, L)] = ninf

        @pl.loop(0, nch)
        def _chunk(c):
            ebase = c * cd
            pltpu.sync_copy(dst_hbm.at[pl.ds(ebase, cd)], dbuf)

            def _scan(v, mcount):
                dv = dbuf[pl.ds(v * L, L)]
                row = dv - node_base
                mask = (row >= 0) & (row < rt)
                pf = jnp.where(mask, jnp.int32(1), jnp.int32(0))
                for sft in (1, 2, 4, 8):
                    idx = jnp.maximum(iota - sft, 0)
                    sh = pf.at[idx].get(mode="promise_in_bounds")
                    pf = jnp.where(iota >= sft, pf + sh, pf)
                cnt = pf[L - 1]

                @pl.when(cnt > 0)
                def _compact():
                    tgt = iota + 1
                    low = jnp.zeros((L,), jnp.int32)
                    for s in (8, 4, 2, 1):
                        t = low + s
                        pv = pf.at[t - 1].get(mode="promise_in_bounds")
                        low = jnp.where(pv < tgt, t, low)
                    eid = ebase + v * L + iota
                    val = (eid << 9) | (row & 511)
                    valc = val.at[jnp.minimum(low, L - 1)].get(
                        mode="promise_in_bounds")
                    mval[pl.ds(mcount, L)] = valc

                return mcount + cnt

            mcount = pl.loop(0, nvec, init_carry=jnp.int32(0))(_scan)
            nsub = (mcount + gb - 1) // gb

            @pl.loop(0, nsub)
            def _sub(g):
                off = g * gb
                for u in range(gb // L):
                    pos = off + u * L + iota
                    v = mval[pl.ds(off + u * L, L)]
                    eid = lax.shift_right_logical(v, 9)
                    gidx[pl.ds(u * L, L)] = jnp.where(pos < mcount, eid, 0)

                pltpu.async_copy(msg_hbm.at[gidx], rows, sem).wait()
                s = jnp.minimum(gb, mcount - off)

                @pl.loop(0, s)
                def _rmw(t):
                    r = mval[pl.ds(off + t, L)][0] & 511
                    for j in range(d // L):
                        sl = pl.ds(j * L, L)
                        acc[r, sl] = jnp.maximum(acc[r, sl], rows[t, sl])

        @pl.when(wid < NW - 1)
        def _store_full():
            pltpu.sync_copy(acc, agg_hbm.at[pl.ds(node_base, rt)])

        @pl.when(wid == NW - 1)
        def _store_last():
            pltpu.sync_copy(acc.at[pl.ds(0, rt_last)],
                            agg_hbm.at[pl.ds(node_base, rt_last)])

    return k(dst, MSG)


def _fin_body(a_ref, o_ref):
    a = a_ref[...]
    o_ref[...] = jnp.tanh(jnp.where(jnp.isfinite(a), a, 0.0))


def _tc_finalize(agg):
    n, d = agg.shape
    blk = 2000
    grid = n // blk
    return pl.pallas_call(
        _fin_body,
        grid=(grid,),
        in_specs=[pl.BlockSpec((blk, d), lambda i: (i, 0))],
        out_specs=pl.BlockSpec((blk, d), lambda i: (i, 0)),
        out_shape=jax.ShapeDtypeStruct((n, d), jnp.float32),
    )(agg)


def kernel(x, edge_index, W1, b1, W2, b2):
    n, d = x.shape
    e = edge_index.shape[1]
    assert e % (NW * 400) == 0 and n % 1000 == 0

    src = edge_index[0]
    dst = edge_index[1]

    P, Q = _tc_pq(x, W1, b1)
    G = _sc_gather_add(dst, src, P, Q)
    MSG = _tc_msg(G, W2, b2)
    AGG = _sc_scatter_max(dst, MSG, n)
    return _tc_finalize(AGG)
